# Initial kernel scaffold; baseline (speedup 1.0000x reference)
import functools

import jax
import jax.numpy as jnp
from jax import lax
from jax.experimental import pallas as pl
from jax.experimental.pallas import tpu as pltpu
from jax.experimental.pallas import tpu_sc as plsc

NC = 2
NS = 16
CH = 16
EPS = 1e-12


def _norm_cols(h):
    n = jnp.sqrt(jnp.sum(h * h, axis=1, keepdims=True))
    return jnp.maximum(n, EPS)


def _prep0_body(beta_ref, x_ref, g_ref, invn_ref):
    x = x_ref[...]
    nc = _norm_cols(x)
    g_ref[...] = (beta_ref[0, 0] / nc) * x
    invn_ref[...] = 1.0 / nc


def _prep_mid_body(beta_ref, u_ref, h_ref, g_ref, invn_ref, *, d):
    u0 = u_ref[0]
    u1 = u_ref[1]
    hs = u0[:, :d] + u1[:, :d]
    s = u0[:, d:d + 1] + u1[:, d:d + 1]
    h = jnp.tanh(hs / (s + EPS))
    nc = _norm_cols(h)
    h_ref[...] = h
    g_ref[...] = (beta_ref[0, 0] / nc) * h
    invn_ref[...] = 1.0 / nc


def _prep3_body(beta_ref, u_ref, w_ref, h_ref, g_ref, invn_ref, *, d, pad):
    u0 = u_ref[0]
    u1 = u_ref[1]
    hs = u0[:, :d] + u1[:, :d]
    s = u0[:, d:d + 1] + u1[:, d:d + 1]
    h = jnp.tanh(hs / (s + EPS))
    p = lax.dot_general(h, w_ref[...], (((1,), (1,)), ((), ())),
                        preferred_element_type=jnp.float32)
    pp = jnp.concatenate([p, jnp.zeros((p.shape[0], pad), jnp.float32)], axis=1)
    nc = _norm_cols(pp)
    h_ref[...] = pp
    g_ref[...] = (beta_ref[0, 0] / nc) * pp
    invn_ref[...] = 1.0 / nc


def _final_body(u_ref, o_ref, *, c, dpad):
    u0 = u_ref[0]
    u1 = u_ref[1]
    hs = u0[:, :c] + u1[:, :c]
    s = u0[:, dpad:dpad + 1] + u1[:, dpad:dpad + 1]
    o_ref[...] = hs / (s + EPS)


@functools.lru_cache(maxsize=None)
def _make_prep0(n, d):
    f32 = jnp.float32
    return pl.pallas_call(
        _prep0_body,
        out_shape=(jax.ShapeDtypeStruct((n, d), f32),
                   jax.ShapeDtypeStruct((n, 1), f32)),
        in_specs=[pl.BlockSpec(memory_space=pltpu.SMEM),
                  pl.BlockSpec(memory_space=pltpu.VMEM)],
    )


@functools.lru_cache(maxsize=None)
def _make_prep_mid(n, d, wrow):
    f32 = jnp.float32
    return pl.pallas_call(
        functools.partial(_prep_mid_body, d=d),
        out_shape=(jax.ShapeDtypeStruct((n, d), f32),
                   jax.ShapeDtypeStruct((n, d), f32),
                   jax.ShapeDtypeStruct((n, 1), f32)),
        in_specs=[pl.BlockSpec(memory_space=pltpu.SMEM),
                  pl.BlockSpec(memory_space=pltpu.VMEM)],
    )


@functools.lru_cache(maxsize=None)
def _make_prep3(n, d, wrow, c, cpad):
    f32 = jnp.float32
    return pl.pallas_call(
        functools.partial(_prep3_body, d=d, pad=cpad - c),
        out_shape=(jax.ShapeDtypeStruct((n, cpad), f32),
                   jax.ShapeDtypeStruct((n, cpad), f32),
                   jax.ShapeDtypeStruct((n, 1), f32)),
        in_specs=[pl.BlockSpec(memory_space=pltpu.SMEM),
                  pl.BlockSpec(memory_space=pltpu.VMEM),
                  pl.BlockSpec(memory_space=pltpu.VMEM)],
    )


@functools.lru_cache(maxsize=None)
def _make_final(n, c, cpad):
    f32 = jnp.float32
    return pl.pallas_call(
        functools.partial(_final_body, c=c, dpad=cpad),
        out_shape=jax.ShapeDtypeStruct((n, c), f32),
        in_specs=[pl.BlockSpec(memory_space=pltpu.VMEM)],
    )


@functools.lru_cache(maxsize=None)
def _make_sc_conv(n_nodes, n_edges, d):
    f32 = jnp.float32
    i32 = jnp.int32
    wrow = d + 16
    nw = NC * NS
    assert n_edges % (nw * CH) == 0
    n_chunks = n_edges // (nw * CH)
    assert n_nodes % NS == 0
    rpt = n_nodes // NS
    zrows = 25
    assert rpt % zrows == 0
    nseg = d // 16

    mesh = plsc.VectorSubcoreMesh(core_axis_name="c", subcore_axis_name="s")

    @functools.partial(
        pl.kernel,
        out_type=jax.ShapeDtypeStruct((NC, n_nodes, wrow), f32),
        mesh=mesh,
        scratch_types=[
            pltpu.VMEM((n_chunks, CH), i32),
            pltpu.VMEM((n_chunks, CH), i32),
            pltpu.VMEM((n_nodes,), f32),
            pltpu.VMEM((CH, d), f32),
            pltpu.VMEM((CH, d), f32),
            pltpu.VMEM((CH, wrow), f32),
            pltpu.VMEM((25, wrow), f32),
            pltpu.VMEM_SHARED((n_nodes, wrow), f32),
            pltpu.SemaphoreType.DMA,
        ],
    )
    def conv(h_hbm, g_hbm, invn_hbm, src_hbm, dst_hbm, u_out,
             src_v, dst_v, invn_v, rs, rd, orow, zbuf, u_sh, sem):
        cid = lax.axis_index("c")
        sid = lax.axis_index("s")
        wid = sid * NC + cid

        pltpu.sync_copy(src_hbm.at[wid], src_v)
        pltpu.sync_copy(dst_hbm.at[wid], dst_v)
        pltpu.sync_copy(invn_hbm, invn_v)

        zero16 = jnp.zeros((16,), f32)
        for r in range(zrows):
            for k in range(wrow // 16):
                zbuf[r, k * 16:(k + 1) * 16] = zero16
        for b in range(rpt // zrows):
            pltpu.sync_copy(zbuf, u_sh.at[pl.ds(sid * rpt + b * zrows, zrows)])
        plsc.subcore_barrier()

        lane = lax.iota(i32, (16,))

        @pl.loop(0, n_chunks)
        def _chunk(c):
            gs = pltpu.async_copy(h_hbm.at[src_v.at[c]], rs, sem)
            gd = pltpu.async_copy(g_hbm.at[dst_v.at[c]], rd, sem)
            gs.wait()
            gd.wait()
            for j in range(CH):
                sj = src_v[c, j]
                invn_j = invn_v[sj]
                acc = rs[j, 0:16] * rd[j, 0:16]
                for k in range(1, nseg):
                    acc = acc + rs[j, k * 16:(k + 1) * 16] * rd[j, k * 16:(k + 1) * 16]
                t = jnp.sum(acc)
                ee = jnp.exp(jnp.full((16,), t, f32)
                             * jnp.full((16,), invn_j, f32) - 1.0)
                orow[j, d:d + 16] = jnp.where(lane == 0, ee, zero16)
                for k in range(nseg):
                    orow[j, k * 16:(k + 1) * 16] = rs[j, k * 16:(k + 1) * 16] * ee
            pltpu.sync_copy(orow, u_sh.at[dst_v.at[c]], add=True)

        plsc.subcore_barrier()
        pltpu.sync_copy(u_sh.at[pl.ds(sid * rpt, rpt)],
                        u_out.at[cid, pl.ds(sid * rpt, rpt)])

    return conv


def kernel(features, edge_index, betas, W):
    n, d = features.shape
    c_out, _ = W.shape
    e = edge_index.shape[1]
    cpad = 48
    nw = NC * NS

    src = edge_index[0].reshape(nw, e // (nw * CH), CH)
    dst = edge_index[1].reshape(nw, e // (nw * CH), CH)

    conv_d = _make_sc_conv(n, e, d)
    conv_c = _make_sc_conv(n, e, cpad)
    prep0 = _make_prep0(n, d)
    prep_mid = _make_prep_mid(n, d, d + 16)
    prep3 = _make_prep3(n, d, d + 16, c_out, cpad)
    final = _make_final(n, c_out, cpad)

    b0 = betas[0].reshape(1, 1)
    g, invn = prep0(b0, features)
    u = conv_d(features, g, invn.reshape(n), src, dst)
    for i in (1, 2):
        h, g, invn = prep_mid(betas[i].reshape(1, 1), u)
        u = conv_d(h, g, invn.reshape(n), src, dst)
    h3, g3, invn3 = prep3(betas[3].reshape(1, 1), u, W)
    u3 = conv_c(h3, g3, invn3.reshape(n), src, dst)
    return final(u3)


# speedup vs baseline: 8.0169x; 8.0169x over previous
import functools

import jax
import jax.numpy as jnp
from jax import lax
from jax.experimental import pallas as pl
from jax.experimental.pallas import tpu as pltpu
from jax.experimental.pallas import tpu_sc as plsc

NC = 2
NS = 16
CH = 16
EPS = 1e-12


def _norm_cols(h):
    n = jnp.sqrt(jnp.sum(h * h, axis=1, keepdims=True))
    return jnp.maximum(n, EPS)


def _prep0_body(beta_ref, x_ref, g_ref, invn_ref):
    x = x_ref[...]
    nc = _norm_cols(x)
    g_ref[...] = (beta_ref[0, 0] / nc) * x
    invn_ref[...] = 1.0 / nc


def _prep_mid_body(beta_ref, u_ref, h_ref, g_ref, invn_ref, *, d, n):
    u0 = u_ref[0]
    u1 = u_ref[1]
    hs = u0[:n, :d] + u1[:n, :d]
    s = u0[:n, d:d + 1] + u1[:n, d:d + 1]
    h = jnp.tanh(hs / (s + EPS))
    nc = _norm_cols(h)
    h_ref[...] = h
    g_ref[...] = (beta_ref[0, 0] / nc) * h
    invn_ref[...] = 1.0 / nc


def _prep3_body(beta_ref, u_ref, w_ref, h_ref, g_ref, invn_ref, *, d, pad, n):
    u0 = u_ref[0]
    u1 = u_ref[1]
    hs = u0[:n, :d] + u1[:n, :d]
    s = u0[:n, d:d + 1] + u1[:n, d:d + 1]
    h = jnp.tanh(hs / (s + EPS))
    p = lax.dot_general(h, w_ref[...], (((1,), (1,)), ((), ())),
                        preferred_element_type=jnp.float32)
    pp = jnp.concatenate([p, jnp.zeros((p.shape[0], pad), jnp.float32)], axis=1)
    nc = _norm_cols(pp)
    h_ref[...] = pp
    g_ref[...] = (beta_ref[0, 0] / nc) * pp
    invn_ref[...] = 1.0 / nc


def _final_body(u_ref, o_ref, *, c, dpad, n):
    u0 = u_ref[0]
    u1 = u_ref[1]
    hs = u0[:n, :c] + u1[:n, :c]
    s = u0[:n, dpad:dpad + 1] + u1[:n, dpad:dpad + 1]
    o_ref[...] = hs / (s + EPS)


@functools.lru_cache(maxsize=None)
def _make_prep0(n, d):
    f32 = jnp.float32
    return pl.pallas_call(
        _prep0_body,
        out_shape=(jax.ShapeDtypeStruct((n, d), f32),
                   jax.ShapeDtypeStruct((n, 1), f32)),
        in_specs=[pl.BlockSpec(memory_space=pltpu.SMEM),
                  pl.BlockSpec(memory_space=pltpu.VMEM)],
    )


@functools.lru_cache(maxsize=None)
def _make_prep_mid(n, d, wrow):
    f32 = jnp.float32
    return pl.pallas_call(
        functools.partial(_prep_mid_body, d=d, n=n),
        out_shape=(jax.ShapeDtypeStruct((n, d), f32),
                   jax.ShapeDtypeStruct((n, d), f32),
                   jax.ShapeDtypeStruct((n, 1), f32)),
        in_specs=[pl.BlockSpec(memory_space=pltpu.SMEM),
                  pl.BlockSpec(memory_space=pltpu.VMEM)],
    )


@functools.lru_cache(maxsize=None)
def _make_prep3(n, d, wrow, c, cpad):
    f32 = jnp.float32
    return pl.pallas_call(
        functools.partial(_prep3_body, d=d, pad=cpad - c, n=n),
        out_shape=(jax.ShapeDtypeStruct((n, cpad), f32),
                   jax.ShapeDtypeStruct((n, cpad), f32),
                   jax.ShapeDtypeStruct((n, 1), f32)),
        in_specs=[pl.BlockSpec(memory_space=pltpu.SMEM),
                  pl.BlockSpec(memory_space=pltpu.VMEM),
                  pl.BlockSpec(memory_space=pltpu.VMEM)],
    )


@functools.lru_cache(maxsize=None)
def _make_final(n, c, cpad):
    f32 = jnp.float32
    return pl.pallas_call(
        functools.partial(_final_body, c=c, dpad=cpad, n=n),
        out_shape=jax.ShapeDtypeStruct((n, c), f32),
        in_specs=[pl.BlockSpec(memory_space=pltpu.VMEM)],
    )


@functools.lru_cache(maxsize=None)
def _make_sc_conv(n_nodes, n_edges, d):
    f32 = jnp.float32
    i32 = jnp.int32
    wrow = d + 16
    nw = NC * NS
    assert n_edges % (nw * CH) == 0
    n_chunks = n_edges // (nw * CH)
    rpt = (-(-n_nodes // NS) + 7) // 8 * 8
    npad = rpt * NS
    nseg = d // 16

    mesh = plsc.VectorSubcoreMesh(core_axis_name="c", subcore_axis_name="s")

    @functools.partial(
        pl.kernel,
        out_type=jax.ShapeDtypeStruct((NC, npad, wrow), f32),
        mesh=mesh,
        compiler_params=pltpu.CompilerParams(use_tc_tiling_on_sc=False,
                                             needs_layout_passes=False),
        scratch_types=[
            pltpu.VMEM((n_chunks, CH), i32),
            pltpu.VMEM((n_chunks, CH), i32),
            pltpu.VMEM((n_nodes,), f32),
            pltpu.VMEM((CH, d), f32),
            pltpu.VMEM((CH, d), f32),
            pltpu.VMEM((CH, wrow), f32),
            pltpu.VMEM((8, wrow), f32),
            pltpu.VMEM_SHARED((npad, wrow), f32),
            pltpu.SemaphoreType.DMA,
        ],
    )
    def conv(h_hbm, g_hbm, invn_hbm, src_hbm, dst_hbm, u_out,
             src_v, dst_v, invn_v, rs, rd, orow, zbuf, u_sh, sem):
        cid = lax.axis_index("c")
        sid = lax.axis_index("s")
        wid = sid * NC + cid

        pltpu.sync_copy(src_hbm.at[wid], src_v)
        pltpu.sync_copy(dst_hbm.at[wid], dst_v)
        pltpu.sync_copy(invn_hbm, invn_v)

        zero16 = jnp.zeros((16,), f32)
        for r in range(8):
            for k in range(wrow // 16):
                zbuf[r, k * 16:(k + 1) * 16] = zero16

        @pl.loop(0, rpt // 8)
        def _zero(b):
            pltpu.sync_copy(zbuf, u_sh.at[pl.ds(sid * rpt + b * 8, 8)])

        plsc.subcore_barrier()

        lane = lax.iota(i32, 16)

        @pl.loop(0, n_chunks)
        def _chunk(c):
            gs = pltpu.async_copy(h_hbm.at[src_v.at[c]], rs, sem)
            gd = pltpu.async_copy(g_hbm.at[dst_v.at[c]], rd, sem)
            gs.wait()
            gd.wait()
            sv = src_v[c]
            invn_g = plsc.load_gather(invn_v, [sv])
            for j in range(CH):
                acc = rs[j, 0:16] * rd[j, 0:16]
                for k in range(1, nseg):
                    acc = acc + rs[j, k * 16:(k + 1) * 16] * rd[j, k * 16:(k + 1) * 16]
                t = jnp.sum(acc)
                ee = jnp.exp(jnp.full((16,), t, f32)
                             * jnp.full((16,), invn_g[j], f32) - 1.0)
                orow[j, d:d + 16] = jnp.where(lane == 0, ee, zero16)
                for k in range(nseg):
                    orow[j, k * 16:(k + 1) * 16] = rs[j, k * 16:(k + 1) * 16] * ee
            pltpu.sync_copy(orow, u_sh.at[dst_v.at[c]], add=True)

        plsc.subcore_barrier()
        pltpu.sync_copy(u_sh.at[pl.ds(sid * rpt, rpt)],
                        u_out.at[cid, pl.ds(sid * rpt, rpt)])

    return conv


def kernel(features, edge_index, betas, W):
    n, d = features.shape
    c_out, _ = W.shape
    e = edge_index.shape[1]
    cpad = 48
    nw = NC * NS

    src = edge_index[0].reshape(nw, e // (nw * CH), CH)
    dst = edge_index[1].reshape(nw, e // (nw * CH), CH)

    conv_d = _make_sc_conv(n, e, d)
    conv_c = _make_sc_conv(n, e, cpad)
    prep0 = _make_prep0(n, d)
    prep_mid = _make_prep_mid(n, d, d + 16)
    prep3 = _make_prep3(n, d, d + 16, c_out, cpad)
    final = _make_final(n, c_out, cpad)

    b0 = betas[0].reshape(1, 1)
    g, invn = prep0(b0, features)
    u = conv_d(features, g, invn.reshape(n), src, dst)
    for i in (1, 2):
        h, g, invn = prep_mid(betas[i].reshape(1, 1), u)
        u = conv_d(h, g, invn.reshape(n), src, dst)
    h3, g3, invn3 = prep3(betas[3].reshape(1, 1), u, W)
    u3 = conv_c(h3, g3, invn3.reshape(n), src, dst)
    return final(u3)


# speedup vs baseline: 15.8590x; 1.9782x over previous
import functools

import jax
import jax.numpy as jnp
from jax import lax
from jax.experimental import pallas as pl
from jax.experimental.pallas import tpu as pltpu
from jax.experimental.pallas import tpu_sc as plsc

NC = 2
NS = 16
CH = 16
EPS = 1e-12


def _norm_cols(h):
    n = jnp.sqrt(jnp.sum(h * h, axis=1, keepdims=True))
    return jnp.maximum(n, EPS)


def _with_invn(h, nc):
    inv = jnp.broadcast_to(1.0 / nc, (h.shape[0], 16))
    return jnp.concatenate([h, inv], axis=1)


def _prep0_body(beta_ref, x_ref, h_ref, g_ref):
    x = x_ref[...]
    nc = _norm_cols(x)
    h_ref[...] = _with_invn(x, nc)
    g_ref[...] = (beta_ref[0, 0] / nc) * x


def _prep_mid_body(beta_ref, u_ref, h_ref, g_ref, *, d, n):
    u0 = u_ref[0]
    u1 = u_ref[1]
    hs = u0[:n, :d] + u1[:n, :d]
    s = u0[:n, d:d + 1] + u1[:n, d:d + 1]
    h = jnp.tanh(hs / (s + EPS))
    nc = _norm_cols(h)
    h_ref[...] = _with_invn(h, nc)
    g_ref[...] = (beta_ref[0, 0] / nc) * h


def _prep3_body(beta_ref, u_ref, w_ref, h_ref, g_ref, *, d, pad, n):
    u0 = u_ref[0]
    u1 = u_ref[1]
    hs = u0[:n, :d] + u1[:n, :d]
    s = u0[:n, d:d + 1] + u1[:n, d:d + 1]
    h = jnp.tanh(hs / (s + EPS))
    p = lax.dot_general(h, w_ref[...], (((1,), (1,)), ((), ())),
                        preferred_element_type=jnp.float32)
    pp = jnp.concatenate([p, jnp.zeros((p.shape[0], pad), jnp.float32)], axis=1)
    nc = _norm_cols(pp)
    h_ref[...] = _with_invn(pp, nc)
    g_ref[...] = (beta_ref[0, 0] / nc) * pp


def _final_body(u_ref, o_ref, *, c, dpad, n):
    u0 = u_ref[0]
    u1 = u_ref[1]
    hs = u0[:n, :c] + u1[:n, :c]
    s = u0[:n, dpad:dpad + 1] + u1[:n, dpad:dpad + 1]
    o_ref[...] = hs / (s + EPS)


@functools.lru_cache(maxsize=None)
def _make_prep0(n, d):
    f32 = jnp.float32
    return pl.pallas_call(
        _prep0_body,
        out_shape=(jax.ShapeDtypeStruct((n, d + 16), f32),
                   jax.ShapeDtypeStruct((n, d), f32)),
        in_specs=[pl.BlockSpec(memory_space=pltpu.SMEM),
                  pl.BlockSpec(memory_space=pltpu.VMEM)],
    )


@functools.lru_cache(maxsize=None)
def _make_prep_mid(n, d, wrow):
    f32 = jnp.float32
    return pl.pallas_call(
        functools.partial(_prep_mid_body, d=d, n=n),
        out_shape=(jax.ShapeDtypeStruct((n, d + 16), f32),
                   jax.ShapeDtypeStruct((n, d), f32)),
        in_specs=[pl.BlockSpec(memory_space=pltpu.SMEM),
                  pl.BlockSpec(memory_space=pltpu.VMEM)],
    )


@functools.lru_cache(maxsize=None)
def _make_prep3(n, d, wrow, c, cpad):
    f32 = jnp.float32
    return pl.pallas_call(
        functools.partial(_prep3_body, d=d, pad=cpad - c, n=n),
        out_shape=(jax.ShapeDtypeStruct((n, cpad + 16), f32),
                   jax.ShapeDtypeStruct((n, cpad), f32)),
        in_specs=[pl.BlockSpec(memory_space=pltpu.SMEM),
                  pl.BlockSpec(memory_space=pltpu.VMEM),
                  pl.BlockSpec(memory_space=pltpu.VMEM)],
    )


@functools.lru_cache(maxsize=None)
def _make_final(n, c, cpad):
    f32 = jnp.float32
    return pl.pallas_call(
        functools.partial(_final_body, c=c, dpad=cpad, n=n),
        out_shape=jax.ShapeDtypeStruct((n, c), f32),
        in_specs=[pl.BlockSpec(memory_space=pltpu.VMEM)],
    )


@functools.lru_cache(maxsize=None)
def _make_sc_conv(n_nodes, n_edges, d):
    f32 = jnp.float32
    i32 = jnp.int32
    wrow = d + 16
    nw = NC * NS
    assert n_edges % (nw * CH) == 0
    n_chunks = n_edges // (nw * CH)
    assert n_chunks % 2 == 1
    rpt = (-(-n_nodes // NS) + 7) // 8 * 8
    npad = rpt * NS
    nseg = d // 16

    mesh = plsc.VectorSubcoreMesh(core_axis_name="c", subcore_axis_name="s")

    @functools.partial(
        pl.kernel,
        out_type=jax.ShapeDtypeStruct((NC, npad, wrow), f32),
        mesh=mesh,
        compiler_params=pltpu.CompilerParams(use_tc_tiling_on_sc=False,
                                             needs_layout_passes=False),
        scratch_types=[
            pltpu.VMEM((n_chunks, CH), i32),
            pltpu.VMEM((n_chunks, CH), i32),
            pltpu.VMEM((CH, d + 16), f32),
            pltpu.VMEM((CH, d + 16), f32),
            pltpu.VMEM((CH, d), f32),
            pltpu.VMEM((CH, d), f32),
            pltpu.VMEM((CH, wrow), f32),
            pltpu.VMEM((CH, wrow), f32),
            pltpu.VMEM((8, wrow), f32),
            pltpu.VMEM_SHARED((npad, wrow), f32),
            pltpu.SemaphoreType.DMA,
            pltpu.SemaphoreType.DMA,
            pltpu.SemaphoreType.DMA,
            pltpu.SemaphoreType.DMA,
        ],
    )
    def conv(h_hbm, g_hbm, src_hbm, dst_hbm, u_out,
             src_v, dst_v, rs0, rs1, rd0, rd1, orow0, orow1,
             zbuf, u_sh, sg0, sg1, ss0, ss1):
        cid = lax.axis_index("c")
        sid = lax.axis_index("s")
        wid = sid * NC + cid
        rs = (rs0, rs1)
        rd = (rd0, rd1)
        orow = (orow0, orow1)
        sg = (sg0, sg1)
        ss = (ss0, ss1)

        pltpu.sync_copy(src_hbm.at[wid], src_v)
        pltpu.sync_copy(dst_hbm.at[wid], dst_v)

        zero16 = jnp.zeros((16,), f32)
        for r in range(8):
            for k in range(wrow // 16):
                zbuf[r, k * 16:(k + 1) * 16] = zero16

        @pl.loop(0, rpt // 8)
        def _zero(b):
            pltpu.sync_copy(zbuf, u_sh.at[pl.ds(sid * rpt + b * 8, 8)])

        plsc.subcore_barrier()

        lane = lax.iota(i32, 16)

        def fire_gather(k, b):
            pltpu.async_copy(h_hbm.at[src_v.at[k]], rs[b], sg[b])
            pltpu.async_copy(g_hbm.at[dst_v.at[k]], rd[b], sg[b])

        def wait_gather(b):
            pltpu.make_async_copy(h_hbm.at[src_v.at[0]], rs[b], sg[b]).wait()
            pltpu.make_async_copy(g_hbm.at[dst_v.at[0]], rd[b], sg[b]).wait()

        def wait_scatter(b):
            pltpu.make_async_copy(
                orow[b], u_sh.at[dst_v.at[0]], ss[b]).wait()

        def compute_chunk(c, b):
            for j in range(CH):
                acc = rs[b][j, 0:16] * rd[b][j, 0:16]
                for k in range(1, nseg):
                    acc = acc + (rs[b][j, k * 16:(k + 1) * 16]
                                 * rd[b][j, k * 16:(k + 1) * 16])
                t = jnp.sum(acc)
                ee = jnp.exp(jnp.full((16,), t, f32)
                             * rs[b][j, d:d + 16] - 1.0)
                orow[b][j, d:d + 16] = jnp.where(lane == 0, ee, zero16)
                for k in range(nseg):
                    orow[b][j, k * 16:(k + 1) * 16] = (
                        rs[b][j, k * 16:(k + 1) * 16] * ee)

        def fire_scatter(c, b):
            pltpu.async_copy(orow[b], u_sh.at[dst_v.at[c]], ss[b], add=True)

        fire_gather(0, 0)

        @pl.loop(0, (n_chunks - 1) // 2)
        def _iter(i):
            for b in (0, 1):
                k = 2 * i + b
                fire_gather(k + 1, 1 - b)
                wait_gather(b)

                @pl.when(k >= 2)
                def _drain():
                    wait_scatter(b)

                compute_chunk(k, b)
                fire_scatter(k, b)

        k = n_chunks - 1
        wait_gather(0)
        wait_scatter(0)
        compute_chunk(k, 0)
        fire_scatter(k, 0)
        wait_scatter(0)
        wait_scatter(1)

        plsc.subcore_barrier()
        pltpu.sync_copy(u_sh.at[pl.ds(sid * rpt, rpt)],
                        u_out.at[cid, pl.ds(sid * rpt, rpt)])

    return conv


def kernel(features, edge_index, betas, W):
    n, d = features.shape
    c_out, _ = W.shape
    e = edge_index.shape[1]
    cpad = 48
    nw = NC * NS

    src = edge_index[0].reshape(nw, e // (nw * CH), CH)
    dst = edge_index[1].reshape(nw, e // (nw * CH), CH)

    conv_d = _make_sc_conv(n, e, d)
    conv_c = _make_sc_conv(n, e, cpad)
    prep0 = _make_prep0(n, d)
    prep_mid = _make_prep_mid(n, d, d + 16)
    prep3 = _make_prep3(n, d, d + 16, c_out, cpad)
    final = _make_final(n, c_out, cpad)

    h, g = prep0(betas[0].reshape(1, 1), features)
    u = conv_d(h, g, src, dst)
    for i in (1, 2):
        h, g = prep_mid(betas[i].reshape(1, 1), u)
        u = conv_d(h, g, src, dst)
    h3, g3 = prep3(betas[3].reshape(1, 1), u, W)
    u3 = conv_c(h3, g3, src, dst)
    return final(u3)
